# Initial kernel scaffold; baseline (speedup 1.0000x reference)
import jax
import jax.numpy as jnp
from jax.experimental import pallas as pl


def kernel(x, position_ids, cos_cached, sin_cached):
    raise NotImplementedError("write your pallas kernel here")


# speedup vs baseline: 1.5350x; 1.5350x over previous
import functools

import jax
import jax.numpy as jnp
from jax import lax
from jax.experimental import pallas as pl
from jax.experimental.pallas import tpu as pltpu
from jax.experimental.pallas import tpu_sc as plsc

_CH = 128


@functools.lru_cache(maxsize=None)
def _gather_call(n, v, d):
    info = plsc.get_sparse_core_info()
    nc, ns = info.num_cores, info.num_subcores
    nw = nc * ns
    b_per_w = n // nw
    n_ch = b_per_w // _CH
    mesh = plsc.VectorSubcoreMesh(core_axis_name="c", subcore_axis_name="s")

    @functools.partial(
        pl.kernel,
        mesh=mesh,
        out_type=[
            jax.ShapeDtypeStruct((n, d), jnp.float32),
            jax.ShapeDtypeStruct((n, d), jnp.float32),
        ],
        scratch_types=[
            pltpu.VMEM((n_ch, _CH), jnp.int32),
            pltpu.VMEM((2, _CH, d), jnp.float32),
            pltpu.VMEM((2, _CH, d), jnp.float32),
        ]
        + [pltpu.SemaphoreType.DMA] * 8,
    )
    def k(cos_hbm, sin_hbm, idx_hbm, cos_out, sin_out, idx_v, cbuf, sbuf,
          cg0, cg1, sg0, sg1, cw0, cw1, sw0, sw1):
        cgs, sgs, cws, sws = (cg0, cg1), (sg0, sg1), (cw0, cw1), (sw0, sw1)
        wid = lax.axis_index("s") * nc + lax.axis_index("c")
        base = wid * b_per_w
        pltpu.sync_copy(idx_hbm.at[pl.ds(wid * n_ch, n_ch)], idx_v)

        cg = [None] * n_ch
        sg = [None] * n_ch
        cw = [None] * n_ch
        sw = [None] * n_ch
        for j in range(n_ch):
            slot = j & 1
            if j >= 2:
                cw[j - 2].wait()
                sw[j - 2].wait()
            cg[j] = pltpu.async_copy(cos_hbm.at[idx_v.at[j]], cbuf.at[slot],
                                     cgs[slot])
            sg[j] = pltpu.async_copy(sin_hbm.at[idx_v.at[j]], sbuf.at[slot],
                                     sgs[slot])
            if j >= 1:
                i = j - 1
                islot = i & 1
                cg[i].wait()
                sg[i].wait()
                cw[i] = pltpu.async_copy(
                    cbuf.at[islot], cos_out.at[pl.ds(base + i * _CH, _CH)],
                    cws[islot])
                sw[i] = pltpu.async_copy(
                    sbuf.at[islot], sin_out.at[pl.ds(base + i * _CH, _CH)],
                    sws[islot])
        i = n_ch - 1
        islot = i & 1
        cg[i].wait()
        sg[i].wait()
        cw[i] = pltpu.async_copy(
            cbuf.at[islot], cos_out.at[pl.ds(base + i * _CH, _CH)], cws[islot])
        sw[i] = pltpu.async_copy(
            sbuf.at[islot], sin_out.at[pl.ds(base + i * _CH, _CH)], sws[islot])
        for i in range(max(0, n_ch - 2), n_ch):
            cw[i].wait()
            sw[i].wait()

    return k


def kernel(x, position_ids, cos_cached, sin_cached):
    del x
    b, s = position_ids.shape
    v, d = cos_cached.shape
    n = b * s
    idx = position_ids.reshape(n // _CH, _CH).astype(jnp.int32)
    cos_flat, sin_flat = _gather_call(n, v, d)(cos_cached, sin_cached, idx)
    return cos_flat.reshape(b, s, d), sin_flat.reshape(b, s, d)


# speedup vs baseline: 1.6127x; 1.0507x over previous
import functools

import jax
import jax.numpy as jnp
from jax import lax
from jax.experimental import pallas as pl
from jax.experimental.pallas import tpu as pltpu
from jax.experimental.pallas import tpu_sc as plsc

_CH = 128
_NBUF = 3


@functools.lru_cache(maxsize=None)
def _gather_call(n, v, d):
    info = plsc.get_sparse_core_info()
    nc, ns = info.num_cores, info.num_subcores
    nw = nc * ns
    b_per_w = n // nw
    n_ch = b_per_w // _CH
    nbuf = min(_NBUF, n_ch)
    mesh = plsc.VectorSubcoreMesh(core_axis_name="c", subcore_axis_name="s")

    @functools.partial(
        pl.kernel,
        mesh=mesh,
        out_type=[
            jax.ShapeDtypeStruct((n, d), jnp.float32),
            jax.ShapeDtypeStruct((n, d), jnp.float32),
        ],
        scratch_types=[
            pltpu.VMEM((n_ch, _CH), jnp.int32),
            pltpu.VMEM((nbuf, _CH, d), jnp.float32),
            pltpu.VMEM((nbuf, _CH, d), jnp.float32),
        ]
        + [pltpu.SemaphoreType.DMA] * (4 * nbuf),
    )
    def k(cos_hbm, sin_hbm, idx_hbm, cos_out, sin_out, idx_v, cbuf, sbuf,
          *sems):
        cgs = sems[0:nbuf]
        sgs = sems[nbuf:2 * nbuf]
        cws = sems[2 * nbuf:3 * nbuf]
        sws = sems[3 * nbuf:4 * nbuf]
        wid = lax.axis_index("s") * nc + lax.axis_index("c")
        base = wid * b_per_w
        pltpu.sync_copy(idx_hbm.at[pl.ds(wid * n_ch, n_ch)], idx_v)

        cg = [None] * n_ch
        sg = [None] * n_ch
        cw = [None] * n_ch
        sw = [None] * n_ch
        for j in range(nbuf):
            slot = j % nbuf
            cg[j] = pltpu.async_copy(cos_hbm.at[idx_v.at[j]], cbuf.at[slot],
                                     cgs[slot])
            sg[j] = pltpu.async_copy(sin_hbm.at[idx_v.at[j]], sbuf.at[slot],
                                     sgs[slot])
        for i in range(n_ch):
            slot = i % nbuf
            cg[i].wait()
            cw[i] = pltpu.async_copy(
                cbuf.at[slot], cos_out.at[pl.ds(base + i * _CH, _CH)],
                cws[slot])
            sg[i].wait()
            sw[i] = pltpu.async_copy(
                sbuf.at[slot], sin_out.at[pl.ds(base + i * _CH, _CH)],
                sws[slot])
            j = i + nbuf
            if j < n_ch:
                cw[i].wait()
                sw[i].wait()
                cg[j] = pltpu.async_copy(cos_hbm.at[idx_v.at[j]],
                                         cbuf.at[slot], cgs[slot])
                sg[j] = pltpu.async_copy(sin_hbm.at[idx_v.at[j]],
                                         sbuf.at[slot], sgs[slot])
        for i in range(max(0, n_ch - nbuf), n_ch):
            cw[i].wait()
            sw[i].wait()

    return k


def kernel(x, position_ids, cos_cached, sin_cached):
    del x
    b, s = position_ids.shape
    v, d = cos_cached.shape
    n = b * s
    idx = position_ids.reshape(n // _CH, _CH).astype(jnp.int32)
    cos_flat, sin_flat = _gather_call(n, v, d)(cos_cached, sin_cached, idx)
    return cos_flat.reshape(b, s, d), sin_flat.reshape(b, s, d)
